# baseline (speedup 1.0000x reference)
import functools

import jax
import jax.numpy as jnp
from jax import lax
from jax.experimental import pallas as pl
from jax.experimental.pallas import tpu as pltpu
from jax.experimental.pallas import tpu_sc as plsc

N_NODES_ = 10000
D_ = 128
K_ = 64
E_PER = 320000
E_TOT = 2 * E_PER
NC_, NS_, LANES_ = 2, 16, 16
NW_ = NC_ * NS_
CHUNK_ = 128
NBUF_ = 4
NCHUNK_ = E_TOT // CHUNK_


def _mm_body(x_ref, w_ref, o_ref):
    o_ref[...] = jnp.dot(x_ref[...], w_ref[...],
                         preferred_element_type=jnp.float32)


def _encode(data, W):
    return pl.pallas_call(
        _mm_body,
        out_shape=jax.ShapeDtypeStruct((N_NODES_, K_), jnp.float32),
        grid=(5,),
        in_specs=[
            pl.BlockSpec((N_NODES_ // 5, D_), lambda i: (i, 0)),
            pl.BlockSpec((D_, K_), lambda i: (0, 0)),
        ],
        out_specs=pl.BlockSpec((N_NODES_ // 5, K_), lambda i: (i, 0)),
    )(data, W)


def _sc_scores(z, srcs, dsts):
    mesh = plsc.VectorSubcoreMesh(core_axis_name="c", subcore_axis_name="s")

    @functools.partial(
        pl.kernel,
        mesh=mesh,
        compiler_params=pltpu.CompilerParams(
            needs_layout_passes=False, use_tc_tiling_on_sc=False),
        out_type=jax.ShapeDtypeStruct((E_TOT,), jnp.float32),
        scratch_types=(
            [pltpu.VMEM((CHUNK_,), jnp.int32)] * NBUF_
            + [pltpu.VMEM((CHUNK_,), jnp.int32)] * NBUF_
            + [pltpu.VMEM((CHUNK_, K_), jnp.float32)] * NBUF_
            + [pltpu.VMEM((CHUNK_, K_), jnp.float32)] * NBUF_
            + [pltpu.VMEM((CHUNK_,), jnp.float32)] * NBUF_
            + [pltpu.SemaphoreType.DMA] * NBUF_
        ),
    )
    def k(z_hbm, src_hbm, dst_hbm, out_hbm, *bufs):
        IS = bufs[0:NBUF_]
        ID = bufs[NBUF_:2 * NBUF_]
        RS = bufs[2 * NBUF_:3 * NBUF_]
        RD = bufs[3 * NBUF_:4 * NBUF_]
        SV = bufs[4 * NBUF_:5 * NBUF_]
        SEM = bufs[5 * NBUF_:6 * NBUF_]
        wid = lax.axis_index("s") * NC_ + lax.axis_index("c")

        def do_compute(b):
            def group(g, carry2):
                base = g * LANES_
                lane = lax.iota(jnp.int32, LANES_)
                res = jnp.zeros((LANES_,), jnp.float32)
                for j in range(LANES_):
                    e = base + j
                    acc = (RS[b][e, pl.ds(0, LANES_)]
                           * RD[b][e, pl.ds(0, LANES_)])
                    for q in range(1, K_ // LANES_):
                        acc = acc + (RS[b][e, pl.ds(q * LANES_, LANES_)]
                                     * RD[b][e, pl.ds(q * LANES_, LANES_)])
                    s = jnp.sum(acc)
                    res = jnp.where(lane == j, s, res)
                SV[b][pl.ds(base, LANES_)] = res
                return carry2

            lax.fori_loop(0, CHUNK_ // LANES_, group, 0)

        nrun = NCHUNK_ // NW_ // NBUF_

        def run_body(p, carry):
            offs = [((p * NBUF_ + b) * NW_ + wid) * CHUNK_
                    for b in range(NBUF_)]
            cps = []
            for b in range(NBUF_):
                pltpu.sync_copy(src_hbm.at[pl.ds(offs[b], CHUNK_)], IS[b])
                pltpu.sync_copy(dst_hbm.at[pl.ds(offs[b], CHUNK_)], ID[b])
                cps.append((
                    pltpu.async_copy(z_hbm.at[IS[b]], RS[b], SEM[b]),
                    pltpu.async_copy(z_hbm.at[ID[b]], RD[b], SEM[b]),
                ))
            for b in range(NBUF_):
                cps[b][0].wait()
                cps[b][1].wait()
                do_compute(b)
                pltpu.sync_copy(SV[b], out_hbm.at[pl.ds(offs[b], CHUNK_)])
            return carry

        lax.fori_loop(0, nrun, run_body, 0)

        def tail_body(c, carry):
            off = (c * NW_ + wid) * CHUNK_
            pltpu.sync_copy(src_hbm.at[pl.ds(off, CHUNK_)], IS[0])
            pltpu.sync_copy(dst_hbm.at[pl.ds(off, CHUNK_)], ID[0])
            cp1 = pltpu.async_copy(z_hbm.at[IS[0]], RS[0], SEM[0])
            cp2 = pltpu.async_copy(z_hbm.at[ID[0]], RD[0], SEM[0])
            cp1.wait()
            cp2.wait()
            do_compute(0)
            pltpu.sync_copy(SV[0], out_hbm.at[pl.ds(off, CHUNK_)])
            return carry

        nch = jnp.where(wid < (NCHUNK_ % NW_), NCHUNK_ // NW_ + 1,
                        NCHUNK_ // NW_)
        lax.fori_loop(nrun * NBUF_, nch, tail_body, 0)

    return k(z, srcs, dsts)


def _bce_body(x_ref, o_ref):
    x = x_ref[...]
    rows = lax.broadcasted_iota(jnp.int32, x.shape, 0)
    t = (rows < (E_PER // x.shape[1])).astype(jnp.float32)
    term = jnp.maximum(x, 0.0) - x * t + jnp.log1p(jnp.exp(-jnp.abs(x)))
    o_ref[...] = (jnp.sum(term) * (1.0 / E_TOT)).reshape(1, 1)


def _bce_reduce(scores2d):
    return pl.pallas_call(
        _bce_body,
        out_shape=jax.ShapeDtypeStruct((1, 1), jnp.float32),
    )(scores2d)


def kernel(data, W, edges_pos, edges_neg):
    z = _encode(data, W)
    srcs = jnp.concatenate(
        (edges_pos[0], edges_neg[0])).astype(jnp.int32)
    dsts = jnp.concatenate(
        (edges_pos[1], edges_neg[1])).astype(jnp.int32)
    scores = _sc_scores(z, srcs, dsts)
    cost = _bce_reduce(scores.reshape(E_TOT // D_, D_))
    return cost.reshape(())


# speedup vs baseline: 1.0751x; 1.0751x over previous
import functools

import jax
import jax.numpy as jnp
from jax import lax
from jax.experimental import pallas as pl
from jax.experimental.pallas import tpu as pltpu
from jax.experimental.pallas import tpu_sc as plsc

N_NODES_ = 10000
D_ = 128
K_ = 64
E_PER = 320000
E_TOT = 2 * E_PER
NC_, NS_, LANES_ = 2, 16, 16
NW_ = NC_ * NS_
CHUNK_ = 128
NBUF_ = 2
NCHUNK_ = E_TOT // CHUNK_


def _mm_body(x_ref, w_ref, o_ref):
    o_ref[...] = jnp.dot(x_ref[...], w_ref[...],
                         preferred_element_type=jnp.float32)


def _encode(data, W):
    return pl.pallas_call(
        _mm_body,
        out_shape=jax.ShapeDtypeStruct((N_NODES_, K_), jnp.float32),
        grid=(5,),
        in_specs=[
            pl.BlockSpec((N_NODES_ // 5, D_), lambda i: (i, 0)),
            pl.BlockSpec((D_, K_), lambda i: (0, 0)),
        ],
        out_specs=pl.BlockSpec((N_NODES_ // 5, K_), lambda i: (i, 0)),
    )(data, W)


def _sc_scores(z, ids):
    mesh = plsc.VectorSubcoreMesh(core_axis_name="c", subcore_axis_name="s")

    @functools.partial(
        pl.kernel,
        mesh=mesh,
        compiler_params=pltpu.CompilerParams(
            needs_layout_passes=False, use_tc_tiling_on_sc=False),
        out_type=jax.ShapeDtypeStruct((E_TOT,), jnp.float32),
        scratch_types=(
            [pltpu.VMEM((2 * CHUNK_,), jnp.int32)] * NBUF_
            + [pltpu.VMEM((CHUNK_, K_), jnp.float32)] * NBUF_
            + [pltpu.VMEM((CHUNK_, K_), jnp.float32)] * NBUF_
            + [pltpu.VMEM((CHUNK_,), jnp.float32)] * NBUF_
            + [pltpu.SemaphoreType.DMA] * NBUF_
        ),
    )
    def k(z_hbm, ids_hbm, out_hbm, *bufs):
        IX = bufs[0:NBUF_]
        RS = bufs[NBUF_:2 * NBUF_]
        RD = bufs[2 * NBUF_:3 * NBUF_]
        SV = bufs[3 * NBUF_:4 * NBUF_]
        SEM = bufs[4 * NBUF_:5 * NBUF_]
        wid = lax.axis_index("s") * NC_ + lax.axis_index("c")

        def do_compute(b):
            def group(g, carry2):
                base = g * LANES_
                lane = lax.iota(jnp.int32, LANES_)
                res = jnp.zeros((LANES_,), jnp.float32)
                for j in range(LANES_):
                    e = base + j
                    acc = (RS[b][e, pl.ds(0, LANES_)]
                           * RD[b][e, pl.ds(0, LANES_)])
                    for q in range(1, K_ // LANES_):
                        acc = acc + (RS[b][e, pl.ds(q * LANES_, LANES_)]
                                     * RD[b][e, pl.ds(q * LANES_, LANES_)])
                    s = jnp.sum(acc)
                    res = jnp.where(lane == j, s, res)
                SV[b][pl.ds(base, LANES_)] = res
                return carry2

            lax.fori_loop(0, CHUNK_ // LANES_, group, 0)

        nrun = NCHUNK_ // NW_ // NBUF_

        def run_body(p, carry):
            chunks = [(p * NBUF_ + b) * NW_ + wid for b in range(NBUF_)]
            cps = []
            for b in range(NBUF_):
                pltpu.sync_copy(
                    ids_hbm.at[pl.ds(chunks[b] * 2 * CHUNK_, 2 * CHUNK_)],
                    IX[b])
                cps.append((
                    pltpu.async_copy(
                        z_hbm.at[IX[b].at[pl.ds(0, CHUNK_)]], RS[b], SEM[b]),
                    pltpu.async_copy(
                        z_hbm.at[IX[b].at[pl.ds(CHUNK_, CHUNK_)]], RD[b],
                        SEM[b]),
                ))
            for b in range(NBUF_):
                cps[b][0].wait()
                cps[b][1].wait()
                do_compute(b)
                pltpu.sync_copy(
                    SV[b], out_hbm.at[pl.ds(chunks[b] * CHUNK_, CHUNK_)])
            return carry

        lax.fori_loop(0, nrun, run_body, 0)

        def tail_body(c, carry):
            ch = c * NW_ + wid
            pltpu.sync_copy(
                ids_hbm.at[pl.ds(ch * 2 * CHUNK_, 2 * CHUNK_)], IX[0])
            cp1 = pltpu.async_copy(
                z_hbm.at[IX[0].at[pl.ds(0, CHUNK_)]], RS[0], SEM[0])
            cp2 = pltpu.async_copy(
                z_hbm.at[IX[0].at[pl.ds(CHUNK_, CHUNK_)]], RD[0], SEM[0])
            cp1.wait()
            cp2.wait()
            do_compute(0)
            pltpu.sync_copy(SV[0], out_hbm.at[pl.ds(ch * CHUNK_, CHUNK_)])
            return carry

        nch = jnp.where(wid < (NCHUNK_ % NW_), NCHUNK_ // NW_ + 1,
                        NCHUNK_ // NW_)
        lax.fori_loop(nrun * NBUF_, nch, tail_body, 0)

    return k(z, ids)


def _bce_body(x_ref, o_ref):
    x = x_ref[...]
    rows = lax.broadcasted_iota(jnp.int32, x.shape, 0)
    t = (rows < (E_PER // x.shape[1])).astype(jnp.float32)
    term = jnp.maximum(x, 0.0) - x * t + jnp.log1p(jnp.exp(-jnp.abs(x)))
    o_ref[...] = (jnp.sum(term) * (1.0 / E_TOT)).reshape(1, 1)


def _bce_reduce(scores2d):
    return pl.pallas_call(
        _bce_body,
        out_shape=jax.ShapeDtypeStruct((1, 1), jnp.float32),
    )(scores2d)


def kernel(data, W, edges_pos, edges_neg):
    z = _encode(data, W)
    srcs = jnp.concatenate(
        (edges_pos[0], edges_neg[0])).astype(jnp.int32)
    dsts = jnp.concatenate(
        (edges_pos[1], edges_neg[1])).astype(jnp.int32)
    ids = jnp.stack(
        (srcs.reshape(NCHUNK_, CHUNK_), dsts.reshape(NCHUNK_, CHUNK_)),
        axis=1).reshape(2 * E_TOT)
    scores = _sc_scores(z, ids)
    cost = _bce_reduce(scores.reshape(E_TOT // D_, D_))
    return cost.reshape(())


# speedup vs baseline: 1.3476x; 1.2535x over previous
import functools

import jax
import jax.numpy as jnp
from jax import lax
from jax.experimental import pallas as pl
from jax.experimental.pallas import tpu as pltpu
from jax.experimental.pallas import tpu_sc as plsc

N_NODES_ = 10000
D_ = 128
K_ = 64
E_PER = 320000
E_TOT = 2 * E_PER
NC_, NS_, LANES_ = 2, 16, 16
NW_ = NC_ * NS_
CHUNK_ = 128
NBUF_ = 2
NCHUNK_ = E_TOT // CHUNK_


def _mm_body(x_ref, w_ref, o_ref):
    o_ref[...] = jnp.dot(x_ref[...], w_ref[...],
                         preferred_element_type=jnp.float32
                         ).astype(jnp.bfloat16)


def _encode(data, W):
    return pl.pallas_call(
        _mm_body,
        out_shape=jax.ShapeDtypeStruct((N_NODES_, K_), jnp.bfloat16),
        grid=(5,),
        in_specs=[
            pl.BlockSpec((N_NODES_ // 5, D_), lambda i: (i, 0)),
            pl.BlockSpec((D_, K_), lambda i: (0, 0)),
        ],
        out_specs=pl.BlockSpec((N_NODES_ // 5, K_), lambda i: (i, 0)),
    )(data, W)


def _sc_scores(z, ids):
    mesh = plsc.VectorSubcoreMesh(core_axis_name="c", subcore_axis_name="s")

    @functools.partial(
        pl.kernel,
        mesh=mesh,
        compiler_params=pltpu.CompilerParams(
            needs_layout_passes=False, use_tc_tiling_on_sc=False),
        out_type=jax.ShapeDtypeStruct((E_TOT,), jnp.float32),
        scratch_types=(
            [pltpu.VMEM((2 * CHUNK_,), jnp.int32)] * NBUF_
            + [pltpu.VMEM((CHUNK_, K_), jnp.bfloat16)] * NBUF_
            + [pltpu.VMEM((CHUNK_, K_), jnp.bfloat16)] * NBUF_
            + [pltpu.VMEM((CHUNK_,), jnp.float32)] * NBUF_
            + [pltpu.SemaphoreType.DMA] * NBUF_
        ),
    )
    def k(z_hbm, ids_hbm, out_hbm, *bufs):
        IX = bufs[0:NBUF_]
        RS = bufs[NBUF_:2 * NBUF_]
        RD = bufs[2 * NBUF_:3 * NBUF_]
        SV = bufs[3 * NBUF_:4 * NBUF_]
        SEM = bufs[4 * NBUF_:5 * NBUF_]
        wid = lax.axis_index("s") * NC_ + lax.axis_index("c")

        def do_compute(b):
            def group(g, carry2):
                base = g * LANES_
                lane = lax.iota(jnp.int32, LANES_)
                res = jnp.zeros((LANES_,), jnp.float32)
                for j in range(LANES_):
                    e = base + j
                    acc = jnp.zeros((LANES_,), jnp.float32)
                    for q in range(K_ // (2 * LANES_)):
                        sb = RS[b][e, pl.ds(q * 2 * LANES_, 2 * LANES_)]
                        db = RD[b][e, pl.ds(q * 2 * LANES_, 2 * LANES_)]
                        s0, s1 = plsc.unpack(
                            sb, format=plsc.PackFormat.INTERLEAVED)
                        d0, d1 = plsc.unpack(
                            db, format=plsc.PackFormat.INTERLEAVED)
                        acc = acc + s0 * d0 + s1 * d1
                    s = jnp.sum(acc)
                    res = jnp.where(lane == j, s, res)
                SV[b][pl.ds(base, LANES_)] = res
                return carry2

            lax.fori_loop(0, CHUNK_ // LANES_, group, 0)

        nrun = NCHUNK_ // NW_ // NBUF_

        def run_body(p, carry):
            chunks = [(p * NBUF_ + b) * NW_ + wid for b in range(NBUF_)]
            cps = []
            for b in range(NBUF_):
                pltpu.sync_copy(
                    ids_hbm.at[pl.ds(chunks[b] * 2 * CHUNK_, 2 * CHUNK_)],
                    IX[b])
                cps.append((
                    pltpu.async_copy(
                        z_hbm.at[IX[b].at[pl.ds(0, CHUNK_)]], RS[b], SEM[b]),
                    pltpu.async_copy(
                        z_hbm.at[IX[b].at[pl.ds(CHUNK_, CHUNK_)]], RD[b],
                        SEM[b]),
                ))
            for b in range(NBUF_):
                cps[b][0].wait()
                cps[b][1].wait()
                do_compute(b)
                pltpu.sync_copy(
                    SV[b], out_hbm.at[pl.ds(chunks[b] * CHUNK_, CHUNK_)])
            return carry

        lax.fori_loop(0, nrun, run_body, 0)

        def tail_body(c, carry):
            ch = c * NW_ + wid
            pltpu.sync_copy(
                ids_hbm.at[pl.ds(ch * 2 * CHUNK_, 2 * CHUNK_)], IX[0])
            cp1 = pltpu.async_copy(
                z_hbm.at[IX[0].at[pl.ds(0, CHUNK_)]], RS[0], SEM[0])
            cp2 = pltpu.async_copy(
                z_hbm.at[IX[0].at[pl.ds(CHUNK_, CHUNK_)]], RD[0], SEM[0])
            cp1.wait()
            cp2.wait()
            do_compute(0)
            pltpu.sync_copy(SV[0], out_hbm.at[pl.ds(ch * CHUNK_, CHUNK_)])
            return carry

        nch = jnp.where(wid < (NCHUNK_ % NW_), NCHUNK_ // NW_ + 1,
                        NCHUNK_ // NW_)
        lax.fori_loop(nrun * NBUF_, nch, tail_body, 0)

    return k(z, ids)


def _bce_body(x_ref, o_ref):
    x = x_ref[...]
    rows = lax.broadcasted_iota(jnp.int32, x.shape, 0)
    t = (rows < (E_PER // x.shape[1])).astype(jnp.float32)
    term = jnp.maximum(x, 0.0) - x * t + jnp.log1p(jnp.exp(-jnp.abs(x)))
    o_ref[...] = (jnp.sum(term) * (1.0 / E_TOT)).reshape(1, 1)


def _bce_reduce(scores2d):
    return pl.pallas_call(
        _bce_body,
        out_shape=jax.ShapeDtypeStruct((1, 1), jnp.float32),
    )(scores2d)


def kernel(data, W, edges_pos, edges_neg):
    z = _encode(data, W)
    srcs = jnp.concatenate(
        (edges_pos[0], edges_neg[0])).astype(jnp.int32)
    dsts = jnp.concatenate(
        (edges_pos[1], edges_neg[1])).astype(jnp.int32)
    ids = jnp.stack(
        (srcs.reshape(NCHUNK_, CHUNK_), dsts.reshape(NCHUNK_, CHUNK_)),
        axis=1).reshape(2 * E_TOT)
    scores = _sc_scores(z, ids)
    cost = _bce_reduce(scores.reshape(E_TOT // D_, D_))
    return cost.reshape(())


# speedup vs baseline: 1.3975x; 1.0370x over previous
import functools

import jax
import jax.numpy as jnp
from jax import lax
from jax.experimental import pallas as pl
from jax.experimental.pallas import tpu as pltpu
from jax.experimental.pallas import tpu_sc as plsc

N_NODES_ = 10000
D_ = 128
K_ = 64
E_PER = 320000
E_TOT = 2 * E_PER
NC_, NS_, LANES_ = 2, 16, 16
NW_ = NC_ * NS_
CHUNK_ = 128
NBUF_ = 2
NCHUNK_ = E_TOT // CHUNK_


def _mm_body(x_ref, w_ref, o_ref):
    o_ref[...] = jnp.dot(x_ref[...], w_ref[...],
                         preferred_element_type=jnp.float32
                         ).astype(jnp.bfloat16)


def _encode(data, W):
    return pl.pallas_call(
        _mm_body,
        out_shape=jax.ShapeDtypeStruct((N_NODES_, K_), jnp.bfloat16),
        grid=(5,),
        in_specs=[
            pl.BlockSpec((N_NODES_ // 5, D_), lambda i: (i, 0)),
            pl.BlockSpec((D_, K_), lambda i: (0, 0)),
        ],
        out_specs=pl.BlockSpec((N_NODES_ // 5, K_), lambda i: (i, 0)),
    )(data, W)


def _sc_scores(z, ids):
    mesh = plsc.VectorSubcoreMesh(core_axis_name="c", subcore_axis_name="s")

    @functools.partial(
        pl.kernel,
        mesh=mesh,
        compiler_params=pltpu.CompilerParams(
            needs_layout_passes=False, use_tc_tiling_on_sc=False),
        out_type=jax.ShapeDtypeStruct((E_TOT,), jnp.float32),
        scratch_types=(
            [pltpu.VMEM((2 * CHUNK_,), jnp.int32)] * NBUF_
            + [pltpu.VMEM((CHUNK_, K_), jnp.bfloat16)] * NBUF_
            + [pltpu.VMEM((CHUNK_, K_), jnp.bfloat16)] * NBUF_
            + [pltpu.VMEM((CHUNK_,), jnp.float32)] * NBUF_
            + [pltpu.SemaphoreType.DMA] * NBUF_
        ),
    )
    def k(z_hbm, ids_hbm, out_hbm, *bufs):
        IX = bufs[0:NBUF_]
        RS = bufs[NBUF_:2 * NBUF_]
        RD = bufs[2 * NBUF_:3 * NBUF_]
        SV = bufs[3 * NBUF_:4 * NBUF_]
        SEM = bufs[4 * NBUF_:5 * NBUF_]
        wid = lax.axis_index("s") * NC_ + lax.axis_index("c")

        def do_compute(b):
            def group(g, carry2):
                base = g * LANES_
                lane = lax.iota(jnp.int32, LANES_)
                res = jnp.zeros((LANES_,), jnp.float32)
                for j in range(LANES_):
                    e = base + j
                    p0 = (RS[b][e, pl.ds(0, 2 * LANES_)]
                          * RD[b][e, pl.ds(0, 2 * LANES_)])
                    p1 = (RS[b][e, pl.ds(2 * LANES_, 2 * LANES_)]
                          * RD[b][e, pl.ds(2 * LANES_, 2 * LANES_)])
                    ps = p0 + p1
                    u0, u1 = plsc.unpack(
                        ps, format=plsc.PackFormat.INTERLEAVED)
                    s = jnp.sum(u0 + u1)
                    res = jnp.where(lane == j, s, res)
                SV[b][pl.ds(base, LANES_)] = res
                return carry2

            lax.fori_loop(0, CHUNK_ // LANES_, group, 0)

        nrun = NCHUNK_ // NW_ // NBUF_

        def run_body(p, carry):
            chunks = [(p * NBUF_ + b) * NW_ + wid for b in range(NBUF_)]
            cps = []
            for b in range(NBUF_):
                pltpu.sync_copy(
                    ids_hbm.at[pl.ds(chunks[b] * 2 * CHUNK_, 2 * CHUNK_)],
                    IX[b])
                cps.append((
                    pltpu.async_copy(
                        z_hbm.at[IX[b].at[pl.ds(0, CHUNK_)]], RS[b], SEM[b]),
                    pltpu.async_copy(
                        z_hbm.at[IX[b].at[pl.ds(CHUNK_, CHUNK_)]], RD[b],
                        SEM[b]),
                ))
            for b in range(NBUF_):
                cps[b][0].wait()
                cps[b][1].wait()
                do_compute(b)
                pltpu.sync_copy(
                    SV[b], out_hbm.at[pl.ds(chunks[b] * CHUNK_, CHUNK_)])
            return carry

        lax.fori_loop(0, nrun, run_body, 0)

        def tail_body(c, carry):
            ch = c * NW_ + wid
            pltpu.sync_copy(
                ids_hbm.at[pl.ds(ch * 2 * CHUNK_, 2 * CHUNK_)], IX[0])
            cp1 = pltpu.async_copy(
                z_hbm.at[IX[0].at[pl.ds(0, CHUNK_)]], RS[0], SEM[0])
            cp2 = pltpu.async_copy(
                z_hbm.at[IX[0].at[pl.ds(CHUNK_, CHUNK_)]], RD[0], SEM[0])
            cp1.wait()
            cp2.wait()
            do_compute(0)
            pltpu.sync_copy(SV[0], out_hbm.at[pl.ds(ch * CHUNK_, CHUNK_)])
            return carry

        nch = jnp.where(wid < (NCHUNK_ % NW_), NCHUNK_ // NW_ + 1,
                        NCHUNK_ // NW_)
        lax.fori_loop(nrun * NBUF_, nch, tail_body, 0)

    return k(z, ids)


def _bce_body(x_ref, o_ref):
    x = x_ref[...]
    rows = lax.broadcasted_iota(jnp.int32, x.shape, 0)
    t = (rows < (E_PER // x.shape[1])).astype(jnp.float32)
    term = jnp.maximum(x, 0.0) - x * t + jnp.log1p(jnp.exp(-jnp.abs(x)))
    o_ref[...] = (jnp.sum(term) * (1.0 / E_TOT)).reshape(1, 1)


def _bce_reduce(scores2d):
    return pl.pallas_call(
        _bce_body,
        out_shape=jax.ShapeDtypeStruct((1, 1), jnp.float32),
    )(scores2d)


def kernel(data, W, edges_pos, edges_neg):
    z = _encode(data, W)
    srcs = jnp.concatenate(
        (edges_pos[0], edges_neg[0])).astype(jnp.int32)
    dsts = jnp.concatenate(
        (edges_pos[1], edges_neg[1])).astype(jnp.int32)
    ids = jnp.stack(
        (srcs.reshape(NCHUNK_, CHUNK_), dsts.reshape(NCHUNK_, CHUNK_)),
        axis=1).reshape(2 * E_TOT)
    scores = _sc_scores(z, ids)
    cost = _bce_reduce(scores.reshape(E_TOT // D_, D_))
    return cost.reshape(())


# speedup vs baseline: 1.4026x; 1.0037x over previous
import functools

import jax
import jax.numpy as jnp
from jax import lax
from jax.experimental import pallas as pl
from jax.experimental.pallas import tpu as pltpu
from jax.experimental.pallas import tpu_sc as plsc

N_NODES_ = 10000
D_ = 128
K_ = 64
E_PER = 320000
E_TOT = 2 * E_PER
NC_, NS_, LANES_ = 2, 16, 16
NW_ = NC_ * NS_
CHUNK_ = 128
NBUF_ = 2
NCHUNK_ = E_TOT // CHUNK_


def _mm_body(x_ref, w_ref, o_ref):
    o_ref[...] = jnp.dot(x_ref[...], w_ref[...],
                         preferred_element_type=jnp.float32
                         ).astype(jnp.bfloat16)


def _encode(data, W):
    return pl.pallas_call(
        _mm_body,
        out_shape=jax.ShapeDtypeStruct((N_NODES_, K_), jnp.bfloat16),
        grid=(5,),
        in_specs=[
            pl.BlockSpec((N_NODES_ // 5, D_), lambda i: (i, 0)),
            pl.BlockSpec((D_, K_), lambda i: (0, 0)),
        ],
        out_specs=pl.BlockSpec((N_NODES_ // 5, K_), lambda i: (i, 0)),
    )(data, W)


def _sc_scores(z, ids):
    mesh = plsc.VectorSubcoreMesh(core_axis_name="c", subcore_axis_name="s")

    @functools.partial(
        pl.kernel,
        mesh=mesh,
        compiler_params=pltpu.CompilerParams(
            needs_layout_passes=False, use_tc_tiling_on_sc=False),
        out_type=jax.ShapeDtypeStruct((E_TOT,), jnp.float32),
        scratch_types=(
            [pltpu.VMEM((2 * CHUNK_,), jnp.int32)] * NBUF_
            + [pltpu.VMEM((CHUNK_, K_), jnp.bfloat16)] * NBUF_
            + [pltpu.VMEM((CHUNK_, K_), jnp.bfloat16)] * NBUF_
            + [pltpu.VMEM((CHUNK_,), jnp.float32)] * NBUF_
            + [pltpu.SemaphoreType.DMA] * NBUF_
        ),
    )
    def k(z_hbm, ids_hbm, out_hbm, *bufs):
        IX = bufs[0:NBUF_]
        RS = bufs[NBUF_:2 * NBUF_]
        RD = bufs[2 * NBUF_:3 * NBUF_]
        SV = bufs[3 * NBUF_:4 * NBUF_]
        SEM = bufs[4 * NBUF_:5 * NBUF_]
        wid = lax.axis_index("s") * NC_ + lax.axis_index("c")

        def do_compute(b):
            def group(g, carry2):
                base = g * LANES_
                lane = lax.iota(jnp.int32, LANES_)
                res = jnp.zeros((LANES_,), jnp.float32)
                for j in range(LANES_):
                    e = base + j
                    p0 = (RS[b][e, pl.ds(0, 2 * LANES_)]
                          * RD[b][e, pl.ds(0, 2 * LANES_)])
                    p1 = (RS[b][e, pl.ds(2 * LANES_, 2 * LANES_)]
                          * RD[b][e, pl.ds(2 * LANES_, 2 * LANES_)])
                    a0, a1 = plsc.unpack(
                        p0, format=plsc.PackFormat.INTERLEAVED)
                    b0, b1 = plsc.unpack(
                        p1, format=plsc.PackFormat.INTERLEAVED)
                    s = jnp.sum((a0 + a1) + (b0 + b1))
                    res = jnp.where(lane == j, s, res)
                SV[b][pl.ds(base, LANES_)] = res
                return carry2

            lax.fori_loop(0, CHUNK_ // LANES_, group, 0)

        nrun = NCHUNK_ // NW_ // NBUF_

        def run_body(p, carry):
            chunks = [(p * NBUF_ + b) * NW_ + wid for b in range(NBUF_)]
            cps = []
            for b in range(NBUF_):
                pltpu.sync_copy(
                    ids_hbm.at[pl.ds(chunks[b] * 2 * CHUNK_, 2 * CHUNK_)],
                    IX[b])
                cps.append((
                    pltpu.async_copy(
                        z_hbm.at[IX[b].at[pl.ds(0, CHUNK_)]], RS[b], SEM[b]),
                    pltpu.async_copy(
                        z_hbm.at[IX[b].at[pl.ds(CHUNK_, CHUNK_)]], RD[b],
                        SEM[b]),
                ))
            for b in range(NBUF_):
                cps[b][0].wait()
                cps[b][1].wait()
                do_compute(b)
                pltpu.sync_copy(
                    SV[b], out_hbm.at[pl.ds(chunks[b] * CHUNK_, CHUNK_)])
            return carry

        lax.fori_loop(0, nrun, run_body, 0)

        def tail_body(c, carry):
            ch = c * NW_ + wid
            pltpu.sync_copy(
                ids_hbm.at[pl.ds(ch * 2 * CHUNK_, 2 * CHUNK_)], IX[0])
            cp1 = pltpu.async_copy(
                z_hbm.at[IX[0].at[pl.ds(0, CHUNK_)]], RS[0], SEM[0])
            cp2 = pltpu.async_copy(
                z_hbm.at[IX[0].at[pl.ds(CHUNK_, CHUNK_)]], RD[0], SEM[0])
            cp1.wait()
            cp2.wait()
            do_compute(0)
            pltpu.sync_copy(SV[0], out_hbm.at[pl.ds(ch * CHUNK_, CHUNK_)])
            return carry

        nch = jnp.where(wid < (NCHUNK_ % NW_), NCHUNK_ // NW_ + 1,
                        NCHUNK_ // NW_)
        lax.fori_loop(nrun * NBUF_, nch, tail_body, 0)

    return k(z, ids)


def _bce_body(x_ref, o_ref):
    x = x_ref[...]
    rows = lax.broadcasted_iota(jnp.int32, x.shape, 0)
    t = (rows < (E_PER // x.shape[1])).astype(jnp.float32)
    term = jnp.maximum(x, 0.0) - x * t + jnp.log1p(jnp.exp(-jnp.abs(x)))
    o_ref[...] = (jnp.sum(term) * (1.0 / E_TOT)).reshape(1, 1)


def _bce_reduce(scores2d):
    return pl.pallas_call(
        _bce_body,
        out_shape=jax.ShapeDtypeStruct((1, 1), jnp.float32),
    )(scores2d)


def kernel(data, W, edges_pos, edges_neg):
    z = _encode(data, W)
    srcs = jnp.concatenate(
        (edges_pos[0], edges_neg[0])).astype(jnp.int32)
    dsts = jnp.concatenate(
        (edges_pos[1], edges_neg[1])).astype(jnp.int32)
    ids = jnp.stack(
        (srcs.reshape(NCHUNK_, CHUNK_), dsts.reshape(NCHUNK_, CHUNK_)),
        axis=1).reshape(2 * E_TOT)
    scores = _sc_scores(z, ids)
    cost = _bce_reduce(scores.reshape(E_TOT // D_, D_))
    return cost.reshape(())


# speedup vs baseline: 1.4063x; 1.0026x over previous
import functools

import jax
import jax.numpy as jnp
from jax import lax
from jax.experimental import pallas as pl
from jax.experimental.pallas import tpu as pltpu
from jax.experimental.pallas import tpu_sc as plsc

N_NODES_ = 10000
D_ = 128
K_ = 64
E_PER = 320000
E_TOT = 2 * E_PER
NC_, NS_, LANES_ = 2, 16, 16
NW_ = NC_ * NS_
CHUNK_ = 128
NBUF_ = 2
NCHUNK_ = E_TOT // CHUNK_


def _mm_body(x_ref, w_ref, o_ref):
    o_ref[...] = jnp.dot(x_ref[...], w_ref[...],
                         preferred_element_type=jnp.float32
                         ).astype(jnp.bfloat16)


def _encode(data, W):
    return pl.pallas_call(
        _mm_body,
        out_shape=jax.ShapeDtypeStruct((N_NODES_, K_), jnp.bfloat16),
        grid=(5,),
        in_specs=[
            pl.BlockSpec((N_NODES_ // 5, D_), lambda i: (i, 0)),
            pl.BlockSpec((D_, K_), lambda i: (0, 0)),
        ],
        out_specs=pl.BlockSpec((N_NODES_ // 5, K_), lambda i: (i, 0)),
    )(data, W)


def _sc_scores(z, ids):
    mesh = plsc.VectorSubcoreMesh(core_axis_name="c", subcore_axis_name="s")

    @functools.partial(
        pl.kernel,
        mesh=mesh,
        compiler_params=pltpu.CompilerParams(
            needs_layout_passes=False, use_tc_tiling_on_sc=False),
        out_type=jax.ShapeDtypeStruct((E_TOT,), jnp.float32),
        scratch_types=(
            [pltpu.VMEM((2 * CHUNK_,), jnp.int32)] * NBUF_
            + [pltpu.VMEM((CHUNK_, K_), jnp.bfloat16)] * NBUF_
            + [pltpu.VMEM((CHUNK_, K_), jnp.bfloat16)] * NBUF_
            + [pltpu.VMEM((CHUNK_,), jnp.float32)] * NBUF_
            + [pltpu.SemaphoreType.DMA] * (2 * NBUF_)
        ),
    )
    def k(z_hbm, ids_hbm, out_hbm, *bufs):
        IX = bufs[0:NBUF_]
        RS = bufs[NBUF_:2 * NBUF_]
        RD = bufs[2 * NBUF_:3 * NBUF_]
        SV = bufs[3 * NBUF_:4 * NBUF_]
        SEM = bufs[4 * NBUF_:5 * NBUF_]
        OSEM = bufs[5 * NBUF_:6 * NBUF_]
        wid = lax.axis_index("s") * NC_ + lax.axis_index("c")

        def do_compute(b):
            def group(g, carry2):
                base = g * LANES_
                lane = lax.iota(jnp.int32, LANES_)
                res = jnp.zeros((LANES_,), jnp.float32)
                for j in range(LANES_):
                    e = base + j
                    p0 = (RS[b][e, pl.ds(0, 2 * LANES_)]
                          * RD[b][e, pl.ds(0, 2 * LANES_)])
                    p1 = (RS[b][e, pl.ds(2 * LANES_, 2 * LANES_)]
                          * RD[b][e, pl.ds(2 * LANES_, 2 * LANES_)])
                    a0, a1 = plsc.unpack(
                        p0, format=plsc.PackFormat.INTERLEAVED)
                    b0, b1 = plsc.unpack(
                        p1, format=plsc.PackFormat.INTERLEAVED)
                    s = jnp.sum((a0 + a1) + (b0 + b1))
                    res = jnp.where(lane == j, s, res)
                SV[b][pl.ds(base, LANES_)] = res
                return carry2

            lax.fori_loop(0, CHUNK_ // LANES_, group, 0)

        nrun = NCHUNK_ // NW_ // NBUF_

        def run_body(p, carry):
            chunks = [(p * NBUF_ + b) * NW_ + wid for b in range(NBUF_)]
            cps = []
            for b in range(NBUF_):
                pltpu.sync_copy(
                    ids_hbm.at[pl.ds(chunks[b] * 2 * CHUNK_, 2 * CHUNK_)],
                    IX[b])
                cps.append((
                    pltpu.async_copy(
                        z_hbm.at[IX[b].at[pl.ds(0, CHUNK_)]], RS[b], SEM[b]),
                    pltpu.async_copy(
                        z_hbm.at[IX[b].at[pl.ds(CHUNK_, CHUNK_)]], RD[b],
                        SEM[b]),
                ))
            ocps = []
            for b in range(NBUF_):
                cps[b][0].wait()
                cps[b][1].wait()
                do_compute(b)
                ocps.append(pltpu.async_copy(
                    SV[b], out_hbm.at[pl.ds(chunks[b] * CHUNK_, CHUNK_)],
                    OSEM[b]))
            for b in range(NBUF_):
                ocps[b].wait()
            return carry

        lax.fori_loop(0, nrun, run_body, 0)

        def tail_body(c, carry):
            ch = c * NW_ + wid
            pltpu.sync_copy(
                ids_hbm.at[pl.ds(ch * 2 * CHUNK_, 2 * CHUNK_)], IX[0])
            cp1 = pltpu.async_copy(
                z_hbm.at[IX[0].at[pl.ds(0, CHUNK_)]], RS[0], SEM[0])
            cp2 = pltpu.async_copy(
                z_hbm.at[IX[0].at[pl.ds(CHUNK_, CHUNK_)]], RD[0], SEM[0])
            cp1.wait()
            cp2.wait()
            do_compute(0)
            pltpu.sync_copy(SV[0], out_hbm.at[pl.ds(ch * CHUNK_, CHUNK_)])
            return carry

        nch = jnp.where(wid < (NCHUNK_ % NW_), NCHUNK_ // NW_ + 1,
                        NCHUNK_ // NW_)
        lax.fori_loop(nrun * NBUF_, nch, tail_body, 0)

    return k(z, ids)


def _bce_body(x_ref, o_ref):
    x = x_ref[...]
    rows = lax.broadcasted_iota(jnp.int32, x.shape, 0)
    t = (rows < (E_PER // x.shape[1])).astype(jnp.float32)
    term = jnp.maximum(x, 0.0) - x * t + jnp.log1p(jnp.exp(-jnp.abs(x)))
    o_ref[...] = (jnp.sum(term) * (1.0 / E_TOT)).reshape(1, 1)


def _bce_reduce(scores2d):
    return pl.pallas_call(
        _bce_body,
        out_shape=jax.ShapeDtypeStruct((1, 1), jnp.float32),
    )(scores2d)


def kernel(data, W, edges_pos, edges_neg):
    z = _encode(data, W)
    srcs = jnp.concatenate(
        (edges_pos[0], edges_neg[0])).astype(jnp.int32)
    dsts = jnp.concatenate(
        (edges_pos[1], edges_neg[1])).astype(jnp.int32)
    ids = jnp.stack(
        (srcs.reshape(NCHUNK_, CHUNK_), dsts.reshape(NCHUNK_, CHUNK_)),
        axis=1).reshape(2 * E_TOT)
    scores = _sc_scores(z, ids)
    cost = _bce_reduce(scores.reshape(E_TOT // D_, D_))
    return cost.reshape(())
